# baseline (speedup 1.0000x reference)
import functools

import jax
import jax.numpy as jnp
from jax import lax
from jax.experimental import pallas as pl
from jax.experimental.pallas import tpu as pltpu
from jax.experimental.pallas import tpu_sc as plsc

F32 = jnp.float32
BF = jnp.bfloat16
HIGH = lax.Precision.HIGHEST


def _softplus(x):
    return jnp.where(x > 0, x, 0.0) + jnp.log1p(jnp.exp(-jnp.abs(x)))


def _bdot(a, b):
    return jnp.dot(a.astype(BF), b.astype(BF), preferred_element_type=F32)


def _edge_body(f_ref, w1_ref, b1_ref, w2_ref, b2_ref, o_ref):
    z = jnp.dot(f_ref[...], w1_ref[...], preferred_element_type=F32)
    z = jnp.maximum(z + b1_ref[...], 0.0).astype(BF)
    m = jnp.dot(z, w2_ref[...], preferred_element_type=F32)
    o_ref[...] = jnp.maximum(m + b2_ref[...], 0.0)


def _edge_mlp(feats16, w1, b1, w2, b2, *, block_rows):
    e, k = feats16.shape
    h = w1.shape[1]
    return pl.pallas_call(
        _edge_body,
        grid=(e // block_rows,),
        in_specs=[
            pl.BlockSpec((block_rows, k), lambda i: (i, 0)),
            pl.BlockSpec((k, h), lambda i: (0, 0)),
            pl.BlockSpec((1, h), lambda i: (0, 0)),
            pl.BlockSpec((h, h), lambda i: (0, 0)),
            pl.BlockSpec((1, h), lambda i: (0, 0)),
        ],
        out_specs=pl.BlockSpec((block_rows, h), lambda i: (i, 0)),
        out_shape=jax.ShapeDtypeStruct((e, h), F32),
    )(feats16, w1.astype(BF), b1.reshape(1, h), w2.astype(BF),
      b2.reshape(1, h))


def _node_body(u_ref, n1_ref, nb1_ref, n2_ref, nb2_ref, o_ref):
    u = jnp.dot(u_ref[...], n1_ref[...], preferred_element_type=F32)
    u = jnp.maximum(u + nb1_ref[...], 0.0).astype(BF)
    o_ref[...] = jnp.dot(u, n2_ref[...], preferred_element_type=F32) + nb2_ref[...]


def _node_mlp(u16, n1_w, n1_b, n2_w, n2_b, *, block_rows):
    n, k = u16.shape
    hid = n2_w.shape[0]
    return pl.pallas_call(
        _node_body,
        grid=(n // block_rows,),
        in_specs=[
            pl.BlockSpec((block_rows, k), lambda i: (i, 0)),
            pl.BlockSpec((k, hid), lambda i: (0, 0)),
            pl.BlockSpec((1, hid), lambda i: (0, 0)),
            pl.BlockSpec((hid, hid), lambda i: (0, 0)),
            pl.BlockSpec((1, hid), lambda i: (0, 0)),
        ],
        out_specs=pl.BlockSpec((block_rows, hid), lambda i: (i, 0)),
        out_shape=jax.ShapeDtypeStruct((n, hid), F32),
    )(u16, n1_w.astype(BF), n1_b.reshape(1, hid), n2_w.astype(BF),
      n2_b.reshape(1, hid))


def _pool_body(h_ref, bt_ref, r1a_ref, r1b_ref, r1c_ref, rb1_ref,
               r2_ref, rb2_ref, r3_ref, rb3_ref, o_ref,
               sum_acc, max_acc, cnt_acc, *, num_graphs, num_blocks):
    i = pl.program_id(0)

    @pl.when(i == 0)
    def _init():
        sum_acc[...] = jnp.zeros_like(sum_acc)
        max_acc[...] = jnp.full_like(max_acc, -jnp.inf)
        cnt_acc[...] = jnp.zeros_like(cnt_acc)

    hb = h_ref[...]
    bt2 = bt_ref[...]
    gids = lax.broadcasted_iota(jnp.int32, (bt2.shape[0], num_graphs), 1)
    maskb = bt2 == gids
    mask = maskb.astype(F32)
    dn = (((0,), (0,)), ((), ()))
    sum_acc[...] += lax.dot_general(mask, hb, dn, preferred_element_type=F32,
                                    precision=HIGH)
    cnt_acc[...] += lax.dot_general(mask, jnp.ones_like(hb[:, :128]), dn,
                                    preferred_element_type=F32, precision=HIGH)

    for g in range(num_graphs):
        sel = jnp.where(maskb[:, g:g + 1], hb, -jnp.inf)
        max_acc[g, :] = jnp.maximum(max_acc[g, :], jnp.max(sel, axis=0))

    @pl.when(i == num_blocks - 1)
    def _readout():
        sums = sum_acc[...]
        cnt = cnt_acc[:, :1]
        mean = (sums / jnp.maximum(cnt, 1.0)).astype(BF)
        mx = max_acc[...].astype(BF)
        sums16 = sums.astype(BF)
        r = jnp.dot(mean, r1a_ref[...], preferred_element_type=F32)
        r = r + jnp.dot(mx, r1b_ref[...], preferred_element_type=F32)
        r = r + jnp.dot(sums16, r1c_ref[...], preferred_element_type=F32)
        r = jnp.maximum(r + rb1_ref[...], 0.0).astype(BF)
        r = jnp.maximum(jnp.dot(r, r2_ref[...], preferred_element_type=F32)
                        + rb2_ref[...], 0.0).astype(BF)
        r = jnp.dot(r, r3_ref[...], preferred_element_type=F32) + rb3_ref[...]
        o_ref[...] = _softplus(r)


def _pool_readout(h, batch, params, *, num_graphs, block_rows):
    n, hid = h.shape
    num_blocks = n // block_rows
    bt2 = batch.reshape(n, 1)
    r1 = params['r1_W'].astype(BF)
    r1a, r1b, r1c = r1[:hid], r1[hid:2 * hid], r1[2 * hid:]
    h2 = params['r2_W'].shape[1]
    r3p = jnp.zeros((h2, 128), F32).at[:, 0].set(params['r3_W'][:, 0])
    rb3p = jnp.zeros((1, 128), F32).at[0, 0].set(params['r3_b'][0])
    out = pl.pallas_call(
        functools.partial(_pool_body, num_graphs=num_graphs,
                          num_blocks=num_blocks),
        grid=(num_blocks,),
        in_specs=[
            pl.BlockSpec((block_rows, hid), lambda i: (i, 0)),
            pl.BlockSpec((block_rows, 1), lambda i: (i, 0)),
            pl.BlockSpec((hid, hid), lambda i: (0, 0)),
            pl.BlockSpec((hid, hid), lambda i: (0, 0)),
            pl.BlockSpec((hid, hid), lambda i: (0, 0)),
            pl.BlockSpec((1, hid), lambda i: (0, 0)),
            pl.BlockSpec((hid, h2), lambda i: (0, 0)),
            pl.BlockSpec((1, h2), lambda i: (0, 0)),
            pl.BlockSpec((h2, 128), lambda i: (0, 0)),
            pl.BlockSpec((1, 128), lambda i: (0, 0)),
        ],
        out_specs=pl.BlockSpec((num_graphs, 128), lambda i: (0, 0)),
        out_shape=jax.ShapeDtypeStruct((num_graphs, 128), F32),
        scratch_shapes=[
            pltpu.VMEM((num_graphs, hid), F32),
            pltpu.VMEM((num_graphs, hid), F32),
            pltpu.VMEM((num_graphs, 128), F32),
        ],
    )(h, bt2, r1a, r1b, r1c, params['r1_b'].reshape(1, hid),
      params['r2_W'].astype(BF), params['r2_b'].reshape(1, h2),
      r3p.astype(BF), rb3p)
    return out[:, :1]


_GCHUNK = 200


def _sc_gather16(h, dsts, srcs):
    n, hid = h.shape
    h16 = h.astype(BF)
    ti32 = lax.bitcast_convert_type(h16.reshape(n, hid // 2, 2), jnp.int32)
    gd, gs = _sc_gather2(ti32, dsts, srcs)
    e = dsts.shape[0]
    hd = lax.bitcast_convert_type(gd, BF).reshape(e, hid)
    hs = lax.bitcast_convert_type(gs, BF).reshape(e, hid)
    return hd, hs


def _sc_gather2(table, dsts, srcs):
    n_nodes, hid = table.shape
    e = dsts.shape[0]
    info = plsc.get_sparse_core_info()
    nc, ns = info.num_cores, info.num_subcores
    nw = nc * ns
    b_per_w = e // nw
    nchunks = b_per_w // _GCHUNK
    mesh = plsc.VectorSubcoreMesh(core_axis_name="c", subcore_axis_name="s")

    @functools.partial(
        pl.kernel, mesh=mesh,
        out_type=[jax.ShapeDtypeStruct((e, hid), table.dtype),
                  jax.ShapeDtypeStruct((e, hid), table.dtype)],
        scratch_types=[
            pltpu.VMEM((_GCHUNK,), jnp.int32),
            pltpu.VMEM((_GCHUNK, hid), table.dtype),
            pltpu.VMEM((_GCHUNK,), jnp.int32),
            pltpu.VMEM((_GCHUNK, hid), table.dtype),
            pltpu.SemaphoreType.DMA,
            pltpu.SemaphoreType.DMA,
        ],
    )
    def gk(table_hbm, dst_hbm, src_hbm, outd_hbm, outs_hbm,
           idxd_v, rowsd_v, idxs_v, rowss_v, semd, sems):
        wid = lax.axis_index("s") * nc + lax.axis_index("c")
        base = wid * b_per_w

        def body(ci, _):
            off = base + ci * _GCHUNK
            pltpu.sync_copy(dst_hbm.at[pl.ds(off, _GCHUNK)], idxd_v)
            pltpu.sync_copy(src_hbm.at[pl.ds(off, _GCHUNK)], idxs_v)
            cpd = pltpu.async_copy(table_hbm.at[idxd_v], rowsd_v, semd)
            cps = pltpu.async_copy(table_hbm.at[idxs_v], rowss_v, sems)
            cpd.wait()
            pltpu.sync_copy(rowsd_v, outd_hbm.at[pl.ds(off, _GCHUNK)])
            cps.wait()
            pltpu.sync_copy(rowss_v, outs_hbm.at[pl.ds(off, _GCHUNK)])
            return ()

        lax.fori_loop(0, nchunks, body, ())

    return gk(table, dsts, srcs)


def kernel(x, pos, edge_index, edge_attr, batch, params):
    n_nodes = x.shape[0]
    src_i = edge_index[0]
    dst = edge_index[1]

    h = jax.nn.relu(_bdot(x, params['enc_W']) + params['enc_b'])
    rel = pos[src_i] - pos[dst]
    dist2 = jnp.sum(rel * rel, axis=-1, keepdims=True)
    ea16 = edge_attr.astype(BF)
    d216 = dist2.astype(BF)

    layers = params['layers']
    for lp in layers[:-1]:
        hd16, hs16 = _sc_gather16(h, dst, src_i)
        feats = jnp.concatenate([hd16, hs16, ea16, d216], axis=-1)
        m = jax.nn.relu(jnp.dot(feats, lp['e1_W'].astype(BF),
                                preferred_element_type=F32) + lp['e1_b'])
        m = jax.nn.relu(_bdot(m, lp['e2_W']) + lp['e2_b'])
        aggr = jax.ops.segment_sum(m, dst, num_segments=n_nodes)
        u = jnp.concatenate([h, aggr], axis=-1)
        u = jax.nn.relu(_bdot(u, lp['n1_W']) + lp['n1_b'])
        h = _bdot(u, lp['n2_W']) + lp['n2_b']

    lp = layers[-1]
    hd16, hs16 = _sc_gather16(h, dst, src_i)
    feats = jnp.concatenate([hd16, hs16, ea16, d216], axis=-1)
    m = _edge_mlp(feats, lp['e1_W'], lp['e1_b'], lp['e2_W'], lp['e2_b'],
                  block_rows=2000)
    aggr = jax.ops.segment_sum(m, dst, num_segments=n_nodes)
    u16 = jnp.concatenate([h, aggr], axis=-1).astype(BF)
    h = _node_mlp(u16, lp['n1_W'], lp['n1_b'], lp['n2_W'], lp['n2_b'],
                  block_rows=2000)

    return _pool_readout(h, batch, params, num_graphs=64, block_rows=1000)


# speedup vs baseline: 1.7626x; 1.7626x over previous
import functools

import jax
import jax.numpy as jnp
from jax import lax
from jax.experimental import pallas as pl
from jax.experimental.pallas import tpu as pltpu
from jax.experimental.pallas import tpu_sc as plsc

F32 = jnp.float32
BF = jnp.bfloat16
HIGH = lax.Precision.HIGHEST


def _softplus(x):
    return jnp.where(x > 0, x, 0.0) + jnp.log1p(jnp.exp(-jnp.abs(x)))


def _bdot(a, b):
    return jnp.dot(a.astype(BF), b.astype(BF), preferred_element_type=F32)


def _edge_body(f_ref, w1_ref, b1_ref, w2_ref, b2_ref, o_ref):
    z = jnp.dot(f_ref[...], w1_ref[...], preferred_element_type=F32)
    z = jnp.maximum(z + b1_ref[...], 0.0).astype(BF)
    m = jnp.dot(z, w2_ref[...], preferred_element_type=F32)
    o_ref[...] = jnp.maximum(m + b2_ref[...], 0.0)


def _edge_mlp(feats16, w1, b1, w2, b2, *, block_rows):
    e, k = feats16.shape
    h = w1.shape[1]
    return pl.pallas_call(
        _edge_body,
        grid=(e // block_rows,),
        in_specs=[
            pl.BlockSpec((block_rows, k), lambda i: (i, 0)),
            pl.BlockSpec((k, h), lambda i: (0, 0)),
            pl.BlockSpec((1, h), lambda i: (0, 0)),
            pl.BlockSpec((h, h), lambda i: (0, 0)),
            pl.BlockSpec((1, h), lambda i: (0, 0)),
        ],
        out_specs=pl.BlockSpec((block_rows, h), lambda i: (i, 0)),
        out_shape=jax.ShapeDtypeStruct((e, h), F32),
    )(feats16, w1.astype(BF), b1.reshape(1, h), w2.astype(BF),
      b2.reshape(1, h))


def _node_body(u_ref, n1_ref, nb1_ref, n2_ref, nb2_ref, o_ref):
    u = jnp.dot(u_ref[...], n1_ref[...], preferred_element_type=F32)
    u = jnp.maximum(u + nb1_ref[...], 0.0).astype(BF)
    o_ref[...] = jnp.dot(u, n2_ref[...], preferred_element_type=F32) + nb2_ref[...]


def _node_mlp(u16, n1_w, n1_b, n2_w, n2_b, *, block_rows):
    n, k = u16.shape
    hid = n2_w.shape[0]
    return pl.pallas_call(
        _node_body,
        grid=(n // block_rows,),
        in_specs=[
            pl.BlockSpec((block_rows, k), lambda i: (i, 0)),
            pl.BlockSpec((k, hid), lambda i: (0, 0)),
            pl.BlockSpec((1, hid), lambda i: (0, 0)),
            pl.BlockSpec((hid, hid), lambda i: (0, 0)),
            pl.BlockSpec((1, hid), lambda i: (0, 0)),
        ],
        out_specs=pl.BlockSpec((block_rows, hid), lambda i: (i, 0)),
        out_shape=jax.ShapeDtypeStruct((n, hid), F32),
    )(u16, n1_w.astype(BF), n1_b.reshape(1, hid), n2_w.astype(BF),
      n2_b.reshape(1, hid))


def _pool_body(h_ref, bt_ref, r1a_ref, r1b_ref, r1c_ref, rb1_ref,
               r2_ref, rb2_ref, r3_ref, rb3_ref, o_ref,
               sum_acc, max_acc, cnt_acc, *, num_graphs, num_blocks):
    i = pl.program_id(0)

    @pl.when(i == 0)
    def _init():
        sum_acc[...] = jnp.zeros_like(sum_acc)
        max_acc[...] = jnp.full_like(max_acc, -jnp.inf)
        cnt_acc[...] = jnp.zeros_like(cnt_acc)

    hb = h_ref[...]
    bt2 = bt_ref[...]
    gids = lax.broadcasted_iota(jnp.int32, (bt2.shape[0], num_graphs), 1)
    maskb = bt2 == gids
    mask = maskb.astype(F32)
    dn = (((0,), (0,)), ((), ()))
    sum_acc[...] += lax.dot_general(mask, hb, dn, preferred_element_type=F32,
                                    precision=HIGH)
    cnt_acc[...] += lax.dot_general(mask, jnp.ones_like(hb[:, :128]), dn,
                                    preferred_element_type=F32, precision=HIGH)

    for g in range(num_graphs):
        sel = jnp.where(maskb[:, g:g + 1], hb, -jnp.inf)
        max_acc[g, :] = jnp.maximum(max_acc[g, :], jnp.max(sel, axis=0))

    @pl.when(i == num_blocks - 1)
    def _readout():
        sums = sum_acc[...]
        cnt = cnt_acc[:, :1]
        mean = (sums / jnp.maximum(cnt, 1.0)).astype(BF)
        mx = max_acc[...].astype(BF)
        sums16 = sums.astype(BF)
        r = jnp.dot(mean, r1a_ref[...], preferred_element_type=F32)
        r = r + jnp.dot(mx, r1b_ref[...], preferred_element_type=F32)
        r = r + jnp.dot(sums16, r1c_ref[...], preferred_element_type=F32)
        r = jnp.maximum(r + rb1_ref[...], 0.0).astype(BF)
        r = jnp.maximum(jnp.dot(r, r2_ref[...], preferred_element_type=F32)
                        + rb2_ref[...], 0.0).astype(BF)
        r = jnp.dot(r, r3_ref[...], preferred_element_type=F32) + rb3_ref[...]
        o_ref[...] = _softplus(r)


def _pool_readout(h, batch, params, *, num_graphs, block_rows):
    n, hid = h.shape
    num_blocks = n // block_rows
    bt2 = batch.reshape(n, 1)
    r1 = params['r1_W'].astype(BF)
    r1a, r1b, r1c = r1[:hid], r1[hid:2 * hid], r1[2 * hid:]
    h2 = params['r2_W'].shape[1]
    r3p = jnp.zeros((h2, 128), F32).at[:, 0].set(params['r3_W'][:, 0])
    rb3p = jnp.zeros((1, 128), F32).at[0, 0].set(params['r3_b'][0])
    out = pl.pallas_call(
        functools.partial(_pool_body, num_graphs=num_graphs,
                          num_blocks=num_blocks),
        grid=(num_blocks,),
        in_specs=[
            pl.BlockSpec((block_rows, hid), lambda i: (i, 0)),
            pl.BlockSpec((block_rows, 1), lambda i: (i, 0)),
            pl.BlockSpec((hid, hid), lambda i: (0, 0)),
            pl.BlockSpec((hid, hid), lambda i: (0, 0)),
            pl.BlockSpec((hid, hid), lambda i: (0, 0)),
            pl.BlockSpec((1, hid), lambda i: (0, 0)),
            pl.BlockSpec((hid, h2), lambda i: (0, 0)),
            pl.BlockSpec((1, h2), lambda i: (0, 0)),
            pl.BlockSpec((h2, 128), lambda i: (0, 0)),
            pl.BlockSpec((1, 128), lambda i: (0, 0)),
        ],
        out_specs=pl.BlockSpec((num_graphs, 128), lambda i: (0, 0)),
        out_shape=jax.ShapeDtypeStruct((num_graphs, 128), F32),
        scratch_shapes=[
            pltpu.VMEM((num_graphs, hid), F32),
            pltpu.VMEM((num_graphs, hid), F32),
            pltpu.VMEM((num_graphs, 128), F32),
        ],
    )(h, bt2, r1a, r1b, r1c, params['r1_b'].reshape(1, hid),
      params['r2_W'].astype(BF), params['r2_b'].reshape(1, h2),
      r3p.astype(BF), rb3p)
    return out[:, :1]


_GCHUNK = 200


def _sc_gather2(table, dsts, srcs):
    n_nodes, hid = table.shape
    e = dsts.shape[0]
    info = plsc.get_sparse_core_info()
    nc, ns = info.num_cores, info.num_subcores
    nw = nc * ns
    b_per_w = e // nw
    nchunks = b_per_w // _GCHUNK
    mesh = plsc.VectorSubcoreMesh(core_axis_name="c", subcore_axis_name="s")

    @functools.partial(
        pl.kernel, mesh=mesh,
        out_type=[jax.ShapeDtypeStruct((e, hid), table.dtype),
                  jax.ShapeDtypeStruct((e, hid), table.dtype)],
        scratch_types=[
            pltpu.VMEM((_GCHUNK,), jnp.int32),
            pltpu.VMEM((_GCHUNK, hid), table.dtype),
            pltpu.VMEM((_GCHUNK,), jnp.int32),
            pltpu.VMEM((_GCHUNK, hid), table.dtype),
            pltpu.SemaphoreType.DMA,
            pltpu.SemaphoreType.DMA,
        ],
    )
    def gk(table_hbm, dst_hbm, src_hbm, outd_hbm, outs_hbm,
           idxd_v, rowsd_v, idxs_v, rowss_v, semd, sems):
        wid = lax.axis_index("s") * nc + lax.axis_index("c")
        base = wid * b_per_w

        def body(ci, _):
            off = base + ci * _GCHUNK
            pltpu.sync_copy(dst_hbm.at[pl.ds(off, _GCHUNK)], idxd_v)
            pltpu.sync_copy(src_hbm.at[pl.ds(off, _GCHUNK)], idxs_v)
            cpd = pltpu.async_copy(table_hbm.at[idxd_v], rowsd_v, semd)
            cps = pltpu.async_copy(table_hbm.at[idxs_v], rowss_v, sems)
            cpd.wait()
            pltpu.sync_copy(rowsd_v, outd_hbm.at[pl.ds(off, _GCHUNK)])
            cps.wait()
            pltpu.sync_copy(rowss_v, outs_hbm.at[pl.ds(off, _GCHUNK)])
            return ()

        lax.fori_loop(0, nchunks, body, ())

    return gk(table, dsts, srcs)


def kernel(x, pos, edge_index, edge_attr, batch, params):
    n_nodes = x.shape[0]
    src_i = edge_index[0]
    dst = edge_index[1]

    h = jax.nn.relu(_bdot(x, params['enc_W']) + params['enc_b'])
    rel = pos[src_i] - pos[dst]
    dist2 = jnp.sum(rel * rel, axis=-1, keepdims=True)
    ea16 = edge_attr.astype(BF)
    d216 = dist2.astype(BF)

    layers = params['layers']
    for lp in layers[:-1]:
        hd, hs = _sc_gather2(h, dst, src_i)
        feats = jnp.concatenate([hd.astype(BF), hs.astype(BF), ea16, d216],
                                axis=-1)
        m = jax.nn.relu(jnp.dot(feats, lp['e1_W'].astype(BF),
                                preferred_element_type=F32) + lp['e1_b'])
        m = jax.nn.relu(_bdot(m, lp['e2_W']) + lp['e2_b'])
        aggr = jax.ops.segment_sum(m, dst, num_segments=n_nodes)
        u = jnp.concatenate([h, aggr], axis=-1)
        u = jax.nn.relu(_bdot(u, lp['n1_W']) + lp['n1_b'])
        h = _bdot(u, lp['n2_W']) + lp['n2_b']

    lp = layers[-1]
    hd, hs = _sc_gather2(h, dst, src_i)
    feats = jnp.concatenate([hd.astype(BF), hs.astype(BF), ea16, d216],
                            axis=-1)
    m = _edge_mlp(feats, lp['e1_W'], lp['e1_b'], lp['e2_W'], lp['e2_b'],
                  block_rows=2000)
    aggr = jax.ops.segment_sum(m, dst, num_segments=n_nodes)
    u16 = jnp.concatenate([h, aggr], axis=-1).astype(BF)
    h = _node_mlp(u16, lp['n1_W'], lp['n1_b'], lp['n2_W'], lp['n2_b'],
                  block_rows=2000)

    return _pool_readout(h, batch, params, num_graphs=64, block_rows=1000)
